# baseline (speedup 1.0000x reference)
import functools

import jax
import jax.numpy as jnp
from jax.experimental import pallas as pl
from jax.experimental.pallas import tpu as pltpu

_IMG = 224
_PATCH = 32
_P = 7
_NP = _P * _P
_PATCH_DIM = 3 * _PATCH * _PATCH
_FEAT = 2048
_NCLS_PAD = 1024

_VMEM_LIMIT = 60 * 1024 * 1024


def _fused_kernel(p_ref, tgt_ref, we_ref, be_ref, wf_ref, bf_ref, loss_ref,
                  s_ref, *, imgs):
    rows = imgs * _NP

    h = jnp.dot(p_ref[...], we_ref[...], preferred_element_type=jnp.float32)
    s_ref[...] = jnp.maximum(h + be_ref[...], 0.0)

    row = jax.lax.broadcasted_iota(jnp.int32, (imgs, rows), 0)
    col = jax.lax.broadcasted_iota(jnp.int32, (imgs, rows), 1)
    lo = row * _NP
    pool = jnp.where((col >= lo) & (col < lo + _NP),
                     jnp.float32(1.0 / _NP), jnp.float32(0.0))
    pooled = jnp.dot(pool, s_ref[...], preferred_element_type=jnp.float32)

    feats = pooled.astype(jnp.bfloat16)
    logits = jnp.dot(feats, wf_ref[...],
                     preferred_element_type=jnp.float32) + bf_ref[...]
    m = jnp.max(logits, axis=-1, keepdims=True)
    lse = m + jnp.log(jnp.sum(jnp.exp(logits - m), axis=-1, keepdims=True))
    cls_ids = jax.lax.broadcasted_iota(jnp.int32, logits.shape, 1)
    tgt_logit = jnp.sum(jnp.where(cls_ids == tgt_ref[...], logits, 0.0),
                        axis=-1, keepdims=True)
    loss_ref[...] = lse - tgt_logit


def _forward(patches, target, w_embed, b_embed, w_fc, b_fc, *, imgs):
    n = target.shape[0]
    nb = n // imgs
    rows = imgs * _NP
    body = functools.partial(_fused_kernel, imgs=imgs)
    return pl.pallas_call(
        body,
        out_shape=jax.ShapeDtypeStruct((n, 1), jnp.float32),
        grid=(nb,),
        in_specs=[
            pl.BlockSpec((rows, _PATCH_DIM), lambda i: (i, 0)),
            pl.BlockSpec((imgs, 1), lambda i: (i, 0)),
            pl.BlockSpec((_PATCH_DIM, _FEAT), lambda i: (0, 0)),
            pl.BlockSpec((1, _FEAT), lambda i: (0, 0)),
            pl.BlockSpec((_FEAT, _NCLS_PAD), lambda i: (0, 0)),
            pl.BlockSpec((1, _NCLS_PAD), lambda i: (0, 0)),
        ],
        out_specs=pl.BlockSpec((imgs, 1), lambda i: (i, 0)),
        scratch_shapes=[pltpu.VMEM((rows, _FEAT), jnp.float32)],
        compiler_params=pltpu.CompilerParams(
            dimension_semantics=("parallel",),
            vmem_limit_bytes=_VMEM_LIMIT,
        ),
    )(patches, target.reshape(n, 1), w_embed, b_embed, w_fc, b_fc)


@jax.jit
def kernel(x, target, w_embed, b_embed, w_fc, b_fc):
    n = x.shape[0]
    x = x.astype(jnp.bfloat16)
    patches = x.reshape(n, 3, _P, _PATCH, _P, _PATCH).transpose(0, 2, 4, 1, 3, 5)
    patches = patches.reshape(n * _NP, _PATCH_DIM)

    if n % 16 == 0:
        imgs = 16
    elif n % 8 == 0:
        imgs = 8
    else:
        imgs = min(n, 8)
    n_eff = (n + imgs - 1) // imgs * imgs
    target = target.astype(jnp.int32)
    if n_eff != n:
        patches = jnp.pad(patches, ((0, (n_eff - n) * _NP), (0, 0)))
        target = jnp.pad(target, (0, n_eff - n))
    losses = _forward(patches, target, w_embed, b_embed, w_fc, b_fc, imgs=imgs)
    return jnp.mean(losses[:n, 0])


# speedup vs baseline: 1.0486x; 1.0486x over previous
import functools

import jax
import jax.numpy as jnp
from jax.experimental import pallas as pl
from jax.experimental.pallas import tpu as pltpu

_IMG = 224
_PATCH = 32
_P = 7
_NP = _P * _P
_PATCH_DIM = 3 * _PATCH * _PATCH
_YX = _PATCH * _PATCH
_FEAT = 2048
_NCLS_PAD = 1024

_VMEM_LIMIT = 60 * 1024 * 1024


def _fused_kernel(a0_ref, a1_ref, a2_ref, tgt_ref, we_ref, be_ref, wf_ref,
                  bf_ref, loss_ref, s_ref, *, imgs):
    rows = imgs * _P
    pw = pl.program_id(1)

    a = jnp.concatenate([a0_ref[...], a1_ref[...], a2_ref[...]], axis=-1)
    h = jnp.dot(a, we_ref[...], preferred_element_type=jnp.float32)
    h = jnp.maximum(h + be_ref[...], 0.0)

    @pl.when(pw == 0)
    def _init():
        s_ref[...] = h

    @pl.when(pw > 0)
    def _acc():
        s_ref[...] = s_ref[...] + h

    @pl.when(pw == _P - 1)
    def _epilogue():
        row = jax.lax.broadcasted_iota(jnp.int32, (imgs, rows), 0)
        col = jax.lax.broadcasted_iota(jnp.int32, (imgs, rows), 1)
        lo = row * _P
        pool = jnp.where((col >= lo) & (col < lo + _P),
                         jnp.float32(1.0 / _NP), jnp.float32(0.0))
        pooled = jnp.dot(pool, s_ref[...], preferred_element_type=jnp.float32)

        feats = pooled.astype(jnp.bfloat16)
        logits = jnp.dot(feats, wf_ref[...],
                         preferred_element_type=jnp.float32) + bf_ref[...]
        m = jnp.max(logits, axis=-1, keepdims=True)
        lse = m + jnp.log(jnp.sum(jnp.exp(logits - m), axis=-1, keepdims=True))
        cls_ids = jax.lax.broadcasted_iota(jnp.int32, logits.shape, 1)
        tgt_logit = jnp.sum(jnp.where(cls_ids == tgt_ref[...], logits, 0.0),
                            axis=-1, keepdims=True)
        loss_ref[...] = lse - tgt_logit


def _forward(g, target, w_embed, b_embed, w_fc, b_fc, *, imgs):
    n = target.shape[0]
    nb = n // imgs
    rows = imgs * _P
    body = functools.partial(_fused_kernel, imgs=imgs)

    def _a_spec(c):
        return pl.BlockSpec((rows, _YX), lambda i, pw, c=c: (i, c * _P + pw))

    return pl.pallas_call(
        body,
        out_shape=jax.ShapeDtypeStruct((n, 1), jnp.float32),
        grid=(nb, _P),
        in_specs=[
            _a_spec(0),
            _a_spec(1),
            _a_spec(2),
            pl.BlockSpec((imgs, 1), lambda i, pw: (i, 0)),
            pl.BlockSpec((_PATCH_DIM, _FEAT), lambda i, pw: (0, 0)),
            pl.BlockSpec((1, _FEAT), lambda i, pw: (0, 0)),
            pl.BlockSpec((_FEAT, _NCLS_PAD), lambda i, pw: (0, 0)),
            pl.BlockSpec((1, _NCLS_PAD), lambda i, pw: (0, 0)),
        ],
        out_specs=pl.BlockSpec((imgs, 1), lambda i, pw: (i, 0)),
        scratch_shapes=[pltpu.VMEM((rows, _FEAT), jnp.float32)],
        compiler_params=pltpu.CompilerParams(
            dimension_semantics=("parallel", "arbitrary"),
            vmem_limit_bytes=_VMEM_LIMIT,
        ),
    )(g, g, g, target.reshape(n, 1), w_embed, b_embed, w_fc, b_fc)


@jax.jit
def kernel(x, target, w_embed, b_embed, w_fc, b_fc):
    n = x.shape[0]
    x = x.astype(jnp.bfloat16)
    xt = x.reshape(n, 3, _P, _PATCH, _P, _PATCH).transpose(0, 2, 1, 4, 3, 5)
    g = xt.reshape(n * _P, 3 * _P * _YX)

    if n % 32 == 0:
        imgs = 32
    elif n % 8 == 0:
        imgs = 8
    else:
        imgs = min(n, 8)
    n_eff = (n + imgs - 1) // imgs * imgs
    target = target.astype(jnp.int32)
    if n_eff != n:
        g = jnp.pad(g, ((0, (n_eff - n) * _P), (0, 0)))
        target = jnp.pad(target, (0, n_eff - n))
    losses = _forward(g, target, w_embed, b_embed, w_fc, b_fc, imgs=imgs)
    return jnp.mean(losses[:n, 0])


# speedup vs baseline: 1.0710x; 1.0214x over previous
import functools

import jax
import jax.numpy as jnp
from jax.experimental import pallas as pl
from jax.experimental.pallas import tpu as pltpu

_IMG = 224
_PATCH = 32
_P = 7
_NP = _P * _P
_PATCH_DIM = 3 * _PATCH * _PATCH
_YX = _PATCH * _PATCH
_FEAT = 2048
_NCLS_PAD = 1024

_VMEM_LIMIT = 60 * 1024 * 1024


def _fused_kernel(a0_ref, a1_ref, a2_ref, tgt_ref, we_ref, be_ref, wf_ref,
                  bf_ref, loss_ref, s_ref, *, imgs):
    rows = imgs * _P
    pw = pl.program_id(1)

    a = jnp.concatenate([a0_ref[...], a1_ref[...], a2_ref[...]], axis=-1)
    h = jnp.dot(a, we_ref[...], preferred_element_type=jnp.float32)
    h = jnp.maximum(h + be_ref[...], 0.0)

    @pl.when(pw == 0)
    def _init():
        s_ref[...] = h

    @pl.when(pw > 0)
    def _acc():
        s_ref[...] = s_ref[...] + h

    @pl.when(pw == _P - 1)
    def _epilogue():
        row = jax.lax.broadcasted_iota(jnp.int32, (imgs, rows), 0)
        col = jax.lax.broadcasted_iota(jnp.int32, (imgs, rows), 1)
        lo = row * _P
        pool = jnp.where((col >= lo) & (col < lo + _P),
                         jnp.float32(1.0 / _NP), jnp.float32(0.0))
        pooled = jnp.dot(pool, s_ref[...], preferred_element_type=jnp.float32)

        feats = pooled.astype(jnp.bfloat16)
        logits = jnp.dot(feats, wf_ref[...],
                         preferred_element_type=jnp.float32) + bf_ref[...]
        m = jnp.max(logits, axis=-1, keepdims=True)
        lse = m + jnp.log(jnp.sum(jnp.exp(logits - m), axis=-1, keepdims=True))
        cls_ids = jax.lax.broadcasted_iota(jnp.int32, logits.shape, 1)
        tgt_logit = jnp.sum(jnp.where(cls_ids == tgt_ref[...], logits, 0.0),
                            axis=-1, keepdims=True)
        loss_ref[...] = lse - tgt_logit


def _forward(g, target, w_embed, b_embed, w_fc, b_fc, *, imgs):
    n = target.shape[0]
    nb = n // imgs
    rows = imgs * _P
    body = functools.partial(_fused_kernel, imgs=imgs)

    def _a_spec(c):
        return pl.BlockSpec((rows, _YX), lambda i, pw, c=c: (i, c * _P + pw))

    return pl.pallas_call(
        body,
        out_shape=jax.ShapeDtypeStruct((n, 1), jnp.float32),
        grid=(nb, _P),
        in_specs=[
            _a_spec(0),
            _a_spec(1),
            _a_spec(2),
            pl.BlockSpec((imgs, 1), lambda i, pw: (i, 0)),
            pl.BlockSpec((_PATCH_DIM, _FEAT), lambda i, pw: (0, 0)),
            pl.BlockSpec((1, _FEAT), lambda i, pw: (0, 0)),
            pl.BlockSpec((_FEAT, _NCLS_PAD), lambda i, pw: (0, 0)),
            pl.BlockSpec((1, _NCLS_PAD), lambda i, pw: (0, 0)),
        ],
        out_specs=pl.BlockSpec((imgs, 1), lambda i, pw: (i, 0)),
        scratch_shapes=[pltpu.VMEM((rows, _FEAT), jnp.float32)],
        compiler_params=pltpu.CompilerParams(
            dimension_semantics=("parallel", "arbitrary"),
            vmem_limit_bytes=_VMEM_LIMIT,
        ),
    )(g, g, g, target.reshape(n, 1), w_embed, b_embed, w_fc, b_fc)


@jax.jit
def kernel(x, target, w_embed, b_embed, w_fc, b_fc):
    n = x.shape[0]
    x = x.astype(jnp.bfloat16)
    xt = x.reshape(n, 3, _P, _PATCH, _P, _PATCH).transpose(0, 2, 1, 4, 3, 5)
    g = xt.reshape(n * _P, 3 * _P * _YX)

    if n % 128 == 0:
        imgs = 128
    elif n % 8 == 0:
        imgs = 8
    else:
        imgs = min(n, 8)
    n_eff = (n + imgs - 1) // imgs * imgs
    target = target.astype(jnp.int32)
    if n_eff != n:
        g = jnp.pad(g, ((0, (n_eff - n) * _P), (0, 0)))
        target = jnp.pad(target, (0, n_eff - n))
    losses = _forward(g, target, w_embed, b_embed, w_fc, b_fc, imgs=imgs)
    return jnp.mean(losses[:n, 0])


# speedup vs baseline: 1.0728x; 1.0016x over previous
import functools

import jax
import jax.numpy as jnp
from jax.experimental import pallas as pl
from jax.experimental.pallas import tpu as pltpu

_IMG = 224
_PATCH = 32
_P = 7
_NP = _P * _P
_PATCH_DIM = 3 * _PATCH * _PATCH
_YX = _PATCH * _PATCH
_FEAT = 2048
_NCLS_PAD = 1024

_VMEM_LIMIT = 60 * 1024 * 1024


def _fused_kernel(a0_ref, a1_ref, a2_ref, tgt_ref, we_ref, be_ref, wf_ref,
                  bf_ref, loss_ref, s_ref, *, imgs):
    rows = imgs * _P
    pw = pl.program_id(1)

    a = jnp.concatenate([a0_ref[...], a1_ref[...], a2_ref[...]], axis=-1)
    h = jnp.dot(a, we_ref[...], preferred_element_type=jnp.float32)
    h = jnp.maximum(h + be_ref[...], 0.0)

    @pl.when(pw == 0)
    def _init():
        s_ref[...] = h

    @pl.when(pw > 0)
    def _acc():
        s_ref[...] = s_ref[...] + h

    @pl.when(pw == _P - 1)
    def _epilogue():
        row = jax.lax.broadcasted_iota(jnp.int32, (imgs, rows), 0)
        col = jax.lax.broadcasted_iota(jnp.int32, (imgs, rows), 1)
        lo = row * _P
        pool = jnp.where((col >= lo) & (col < lo + _P),
                         jnp.float32(1.0 / _NP), jnp.float32(0.0))
        pooled = jnp.dot(pool, s_ref[...], preferred_element_type=jnp.float32)

        feats = pooled.astype(jnp.bfloat16)
        logits = jnp.dot(feats, wf_ref[...],
                         preferred_element_type=jnp.float32) + bf_ref[...]
        m = jnp.max(logits, axis=-1, keepdims=True)
        lse = m + jnp.log(jnp.sum(jnp.exp(logits - m), axis=-1, keepdims=True))
        cls_ids = jax.lax.broadcasted_iota(jnp.int32, logits.shape, 1)
        tgt_logit = jnp.sum(jnp.where(cls_ids == tgt_ref[...], logits, 0.0),
                            axis=-1, keepdims=True)
        loss_ref[...] = lse - tgt_logit


def _forward(g, target, w_embed, b_embed, w_fc, b_fc, *, imgs):
    n = target.shape[0]
    nb = n // imgs
    rows = imgs * _P
    body = functools.partial(_fused_kernel, imgs=imgs)

    def _a_spec(c):
        return pl.BlockSpec((rows, _YX), lambda i, pw, c=c: (i, c * _P + pw))

    return pl.pallas_call(
        body,
        out_shape=jax.ShapeDtypeStruct((n, 1), jnp.float32),
        grid=(nb, _P),
        in_specs=[
            _a_spec(0),
            _a_spec(1),
            _a_spec(2),
            pl.BlockSpec((imgs, 1), lambda i, pw: (i, 0)),
            pl.BlockSpec((_PATCH_DIM, _FEAT), lambda i, pw: (0, 0)),
            pl.BlockSpec((1, _FEAT), lambda i, pw: (0, 0)),
            pl.BlockSpec((_FEAT, _NCLS_PAD), lambda i, pw: (0, 0)),
            pl.BlockSpec((1, _NCLS_PAD), lambda i, pw: (0, 0)),
        ],
        out_specs=pl.BlockSpec((imgs, 1), lambda i, pw: (i, 0)),
        scratch_shapes=[pltpu.VMEM((rows, _FEAT), jnp.float32)],
        compiler_params=pltpu.CompilerParams(
            dimension_semantics=("parallel", "arbitrary"),
            vmem_limit_bytes=_VMEM_LIMIT,
        ),
    )(g, g, g, target.reshape(n, 1), w_embed, b_embed, w_fc, b_fc)


@jax.jit
def kernel(x, target, w_embed, b_embed, w_fc, b_fc):
    n = x.shape[0]
    xt = x.reshape(n, 3, _P, _PATCH, _P, _PATCH).transpose(0, 2, 1, 4, 3, 5)
    xt = xt.astype(jnp.bfloat16)
    g = xt.reshape(n * _P, 3 * _P * _YX)

    if n % 128 == 0:
        imgs = 128
    elif n % 8 == 0:
        imgs = 8
    else:
        imgs = min(n, 8)
    n_eff = (n + imgs - 1) // imgs * imgs
    target = target.astype(jnp.int32)
    if n_eff != n:
        g = jnp.pad(g, ((0, (n_eff - n) * _P), (0, 0)))
        target = jnp.pad(target, (0, n_eff - n))
    losses = _forward(g, target, w_embed, b_embed, w_fc, b_fc, imgs=imgs)
    return jnp.mean(losses[:n, 0])
